# Initial kernel scaffold; baseline (speedup 1.0000x reference)
import jax
import jax.numpy as jnp
from jax.experimental import pallas as pl


def kernel(pos):
    raise NotImplementedError("write your pallas kernel here")


# speedup vs baseline: 1.2974x; 1.2974x over previous
import jax
import jax.numpy as jnp
from jax.experimental import pallas as pl

R_MAX = 5.0
N = 4096
BR = 256


def _nl_kernel(prow_ref, pcol_ref, el_ref, mask_ref, nn_ref):
    i = pl.program_id(0)
    d2 = None
    for c in range(3):
        xi = prow_ref[:, c:c + 1]
        xj = pcol_ref[c:c + 1, :]
        d = xi - xj
        d2 = d * d if d2 is None else d2 + d * d
    dist = jnp.sqrt(d2 + 1e-12)
    rid = jax.lax.broadcasted_iota(jnp.int32, (BR, N), 0) + i * BR
    cid = jax.lax.broadcasted_iota(jnp.int32, (BR, N), 1)
    m = (dist <= R_MAX) & (rid != cid)
    el_ref[...] = jnp.where(m, dist, 0.0)
    mask_ref[...] = m
    nn_ref[...] = jnp.sum(m, axis=1, dtype=jnp.int32, keepdims=True)


def kernel(pos):
    pos_t = pos.T
    grid = (N // BR,)
    el, mask, nn = pl.pallas_call(
        _nl_kernel,
        grid=grid,
        in_specs=[
            pl.BlockSpec((BR, 3), lambda i: (i, 0)),
            pl.BlockSpec((3, N), lambda i: (0, 0)),
        ],
        out_specs=[
            pl.BlockSpec((BR, N), lambda i: (i, 0)),
            pl.BlockSpec((BR, N), lambda i: (i, 0)),
            pl.BlockSpec((BR, 1), lambda i: (i, 0)),
        ],
        out_shape=[
            jax.ShapeDtypeStruct((N, N), jnp.float32),
            jax.ShapeDtypeStruct((N, N), jnp.bool_),
            jax.ShapeDtypeStruct((N, 1), jnp.int32),
        ],
    )(pos, pos_t)
    return el, mask, nn.reshape(N)


# speedup vs baseline: 1.3397x; 1.0326x over previous
import jax
import jax.numpy as jnp
from jax.experimental import pallas as pl

R_MAX = 5.0
N = 4096
BR = 256


def _nl_kernel(prow_ref, pcol_ref, el_ref, mask_ref, nn_ref):
    d2 = None
    for c in range(3):
        xi = prow_ref[:, c:c + 1]
        xj = pcol_ref[c:c + 1, :]
        d = xi - xj
        d2 = d * d if d2 is None else d2 + d * d
    m = (d2 <= R_MAX * R_MAX) & (d2 > 0.0)
    el_ref[...] = jnp.sqrt(jnp.where(m, d2, 0.0))
    mask_ref[...] = m
    nn_ref[...] = jnp.sum(m, axis=1, dtype=jnp.int32, keepdims=True)


def kernel(pos):
    pos_t = pos.T
    grid = (N // BR,)
    el, mask, nn = pl.pallas_call(
        _nl_kernel,
        grid=grid,
        in_specs=[
            pl.BlockSpec((BR, 3), lambda i: (i, 0)),
            pl.BlockSpec((3, N), lambda i: (0, 0)),
        ],
        out_specs=[
            pl.BlockSpec((BR, N), lambda i: (i, 0)),
            pl.BlockSpec((BR, N), lambda i: (i, 0)),
            pl.BlockSpec((BR, 1), lambda i: (i, 0)),
        ],
        out_shape=[
            jax.ShapeDtypeStruct((N, N), jnp.float32),
            jax.ShapeDtypeStruct((N, N), jnp.bool_),
            jax.ShapeDtypeStruct((N, 1), jnp.int32),
        ],
    )(pos, pos_t)
    return el, mask, nn.reshape(N)
